# Initial kernel scaffold; baseline (speedup 1.0000x reference)
import jax
import jax.numpy as jnp
from jax.experimental import pallas as pl


def kernel(nodes_features, edge_index, edge_dims_weights, distance_dims_weights):
    raise NotImplementedError("write your pallas kernel here")


# speedup vs baseline: 7.2055x; 7.2055x over previous
import functools

import jax
import jax.numpy as jnp
import numpy as np
from jax import lax
from jax.experimental import pallas as pl
from jax.experimental.pallas import tpu as pltpu
from jax.experimental.pallas import tpu_sc as plsc

NC = 2
NS = 16
NW = NC * NS
CH = 128

_F32 = jnp.float32


def _sel_matrix() -> np.ndarray:
    s = np.zeros((128, 16), dtype=np.float32)
    for l in range(128):
        s[l, 8 * (l // 64) + (l % 64) // 16] = 1.0
    return s


_SEL = _sel_matrix()


def _sc_gather2_body(nchunks, x64, src_i, trg_i, src_out, trg_out,
                     idx_s, idx_t, rows_s, rows_t, sem_s, sem_t):
    wid = lax.axis_index("s") * NC + lax.axis_index("c")
    niter = (nchunks + NW - 1) // NW

    def body(j, carry):
        cidx = wid + j * NW

        @pl.when(cidx < nchunks)
        def _():
            b = pl.multiple_of(cidx * CH, CH)
            pltpu.sync_copy(src_i.at[pl.ds(b, CH)], idx_s)
            pltpu.sync_copy(trg_i.at[pl.ds(b, CH)], idx_t)
            cs = pltpu.async_copy(x64.at[idx_s], rows_s, sem_s)
            ct = pltpu.async_copy(x64.at[idx_t], rows_t, sem_t)
            cs.wait()
            ct.wait()
            pltpu.sync_copy(rows_s, src_out.at[pl.ds(b, CH)])
            pltpu.sync_copy(rows_t, trg_out.at[pl.ds(b, CH)])

        return carry

    lax.fori_loop(0, niter, body, 0)


def _sc_scatter_body(nchunks, n_nodes, exp8, trg_i, zrows, parts,
                     idx, vals, shared):
    cid = lax.axis_index("c")
    sid = lax.axis_index("s")
    wid = sid * NC + cid
    rpt = n_nodes // NS
    niter = (nchunks + NW - 1) // NW

    pltpu.sync_copy(zrows, shared.at[pl.ds(sid * rpt, rpt)])
    plsc.subcore_barrier()

    def body(j, carry):
        cidx = wid + j * NW

        @pl.when(cidx < nchunks)
        def _():
            b = pl.multiple_of(cidx * CH, CH)
            pltpu.sync_copy(trg_i.at[pl.ds(b, CH)], idx)
            pltpu.sync_copy(exp8.at[pl.ds(b, CH)], vals)
            pltpu.sync_copy(vals, shared.at[idx], add=True)

        return carry

    lax.fori_loop(0, niter, body, 0)
    plsc.subcore_barrier()
    pltpu.sync_copy(shared.at[pl.ds(sid * rpt, rpt)],
                    parts.at[cid, pl.ds(sid * rpt, rpt)])


def _sc_gather1_body(nchunks, nbr8, trg_i, out8, idx, rows, sem):
    wid = lax.axis_index("s") * NC + lax.axis_index("c")
    niter = (nchunks + NW - 1) // NW

    def body(j, carry):
        cidx = wid + j * NW

        @pl.when(cidx < nchunks)
        def _():
            b = pl.multiple_of(cidx * CH, CH)
            pltpu.sync_copy(trg_i.at[pl.ds(b, CH)], idx)
            pltpu.async_copy(nbr8.at[idx], rows, sem).wait()
            pltpu.sync_copy(rows, out8.at[pl.ds(b, CH)])

        return carry

    lax.fori_loop(0, niter, body, 0)


def _tc_dist_body(s_ref, t_ref, c_ref, sel_ref, d_ref, sum_ref, min_ref):
    i = pl.program_id(0)
    diff = t_ref[...] - s_ref[...]
    w2 = diff * diff * c_ref[...]
    d16 = jnp.dot(w2, sel_ref[...], preferred_element_type=_F32)
    d_ref[...] = d16
    psum = jnp.sum(d16, axis=0, keepdims=True)
    pmin = jnp.min(d16, axis=0, keepdims=True)

    @pl.when(i == 0)
    def _():
        sum_ref[...] = psum
        min_ref[...] = pmin

    @pl.when(i != 0)
    def _():
        sum_ref[...] += psum
        min_ref[...] = jnp.minimum(min_ref[...], pmin)


def _tc_exp_body(d_ref, mean_ref, m_ref, e_ref):
    x = d_ref[...] + mean_ref[...]
    lk = jnp.where(x >= 0.0, x, 0.2 * x)
    e_ref[...] = jnp.exp(-lk - m_ref[...])


def _tc_add_body(a_ref, b_ref, o_ref):
    o_ref[...] = a_ref[...] + b_ref[...]


def _tc_div_body(e_ref, dn_ref, o_ref):
    o_ref[...] = e_ref[...] / (dn_ref[...] + 1e-16)


def kernel(nodes_features, edge_index, edge_dims_weights, distance_dims_weights):
    n_nodes, n_heads, n_feat = nodes_features.shape
    n_edges = edge_index.shape[1]
    hf = n_heads * n_feat
    assert hf == 64 and n_heads == 4 and n_feat == 16
    assert n_edges % (2 * CH) == 0 and n_nodes % NS == 0

    nchunks = n_edges // CH
    trg = edge_index[0]
    src = edge_index[1]
    x64 = nodes_features.reshape(n_nodes, hf)

    mesh = plsc.VectorSubcoreMesh(core_axis_name="c", subcore_axis_name="s",
                                  num_cores=NC, num_subcores=NS)
    sc_params = pltpu.CompilerParams(use_tc_tiling_on_sc=False)

    gather2 = pl.kernel(
        functools.partial(_sc_gather2_body, nchunks),
        out_type=[jax.ShapeDtypeStruct((n_edges, hf), _F32),
                  jax.ShapeDtypeStruct((n_edges, hf), _F32)],
        mesh=mesh,
        scratch_types=[pltpu.VMEM((CH,), jnp.int32),
                       pltpu.VMEM((CH,), jnp.int32),
                       pltpu.VMEM((CH, hf), _F32),
                       pltpu.VMEM((CH, hf), _F32),
                       pltpu.SemaphoreType.DMA,
                       pltpu.SemaphoreType.DMA],
        compiler_params=sc_params,
    )
    src_rows, trg_rows = gather2(x64, src, trg)

    e2 = n_edges // 2
    be2 = 4000
    assert e2 % be2 == 0
    cw = (distance_dims_weights * edge_dims_weights * edge_dims_weights)
    c128 = jnp.tile(cw.reshape(1, hf), (1, 2))
    d16, sums16, mins16 = pl.pallas_call(
        _tc_dist_body,
        grid=(e2 // be2,),
        in_specs=[pl.BlockSpec((be2, 128), lambda i: (i, 0)),
                  pl.BlockSpec((be2, 128), lambda i: (i, 0)),
                  pl.BlockSpec((1, 128), lambda i: (0, 0)),
                  pl.BlockSpec((128, 16), lambda i: (0, 0))],
        out_specs=[pl.BlockSpec((be2, 16), lambda i: (i, 0)),
                   pl.BlockSpec((1, 16), lambda i: (0, 0)),
                   pl.BlockSpec((1, 16), lambda i: (0, 0))],
        out_shape=[jax.ShapeDtypeStruct((e2, 16), _F32),
                   jax.ShapeDtypeStruct((1, 16), _F32),
                   jax.ShapeDtypeStruct((1, 16), _F32)],
    )(src_rows.reshape(e2, 128), trg_rows.reshape(e2, 128), c128,
      jnp.asarray(_SEL))

    s16 = sums16[0]
    m16 = mins16[0]
    mean8 = (s16[:8] + s16[8:]) / n_edges
    t8 = jnp.minimum(m16[:8], m16[8:]) + mean8
    lk8 = jnp.where(t8 >= 0.0, t8, 0.2 * t8)
    mshift = jnp.max(-lk8)
    mean128 = jnp.tile(mean8, 16).reshape(1, 128)
    m128 = jnp.full((1, 128), mshift, dtype=_F32)

    e16 = n_edges // 16
    be3 = 5000
    assert e16 % be3 == 0
    e128 = pl.pallas_call(
        _tc_exp_body,
        grid=(e16 // be3,),
        in_specs=[pl.BlockSpec((be3, 128), lambda i: (i, 0)),
                  pl.BlockSpec((1, 128), lambda i: (0, 0)),
                  pl.BlockSpec((1, 128), lambda i: (0, 0))],
        out_specs=pl.BlockSpec((be3, 128), lambda i: (i, 0)),
        out_shape=jax.ShapeDtypeStruct((e16, 128), _F32),
    )(d16.reshape(e16, 128), mean128, m128)
    exp8 = e128.reshape(n_edges, 8)

    rpt = n_nodes // NS
    zrows = jnp.zeros((rpt, 8), dtype=_F32)
    scatter = pl.kernel(
        functools.partial(_sc_scatter_body, nchunks, n_nodes),
        out_type=jax.ShapeDtypeStruct((NC, n_nodes, 8), _F32),
        mesh=mesh,
        scratch_types=[pltpu.VMEM((CH,), jnp.int32),
                       pltpu.VMEM((CH, 8), _F32),
                       pltpu.VMEM_SHARED((n_nodes, 8), _F32)],
        compiler_params=sc_params,
    )
    parts = scatter(exp8, trg, zrows)

    nr = n_nodes * 8 // 128
    nbr128 = pl.pallas_call(
        _tc_add_body,
        out_shape=jax.ShapeDtypeStruct((nr, 128), _F32),
    )(parts[0].reshape(nr, 128), parts[1].reshape(nr, 128))

    gatherd = pl.kernel(
        functools.partial(_sc_gather1_body, nchunks),
        out_type=jax.ShapeDtypeStruct((n_edges, 8), _F32),
        mesh=mesh,
        scratch_types=[pltpu.VMEM((CH,), jnp.int32),
                       pltpu.VMEM((CH, 8), _F32),
                       pltpu.SemaphoreType.DMA],
        compiler_params=sc_params,
    )
    denom8 = gatherd(nbr128.reshape(n_nodes, 8), trg)

    att128 = pl.pallas_call(
        _tc_div_body,
        grid=(e16 // be3,),
        in_specs=[pl.BlockSpec((be3, 128), lambda i: (i, 0)),
                  pl.BlockSpec((be3, 128), lambda i: (i, 0))],
        out_specs=pl.BlockSpec((be3, 128), lambda i: (i, 0)),
        out_shape=jax.ShapeDtypeStruct((e16, 128), _F32),
    )(e128, denom8.reshape(e16, 128))

    attentions = att128.reshape(n_edges, 8)[:, :4].reshape(n_edges, n_heads, 1)
    return attentions, src_rows.reshape(n_edges, n_heads, n_feat)
